# Initial kernel scaffold; baseline (speedup 1.0000x reference)
import jax
import jax.numpy as jnp
from jax.experimental import pallas as pl


def kernel(x_nchw, w_dw, w_pw):
    raise NotImplementedError("write your pallas kernel here")


# speedup vs baseline: 4.4390x; 4.4390x over previous
import functools

import jax
import jax.numpy as jnp
from jax.experimental import pallas as pl
from jax.experimental.pallas import tpu as pltpu


def _lshift(x, d):
    if d == 0:
        return x
    n = x.shape[-1]
    zeros = jnp.zeros(x.shape[:-1] + (abs(d),), x.dtype)
    if d > 0:
        return jnp.concatenate([x[..., d:], zeros], axis=-1)
    return jnp.concatenate([zeros, x[..., :n + d]], axis=-1)


def _fused_kernel(wdw_ref, wpw_ref, cmask_ref, x_ref, o_ref, *, k, pad, w):
    x = x_ref[0]

    acc = None
    for kw in range(k):
        dc = kw - pad
        col = x if dc == 0 else _lshift(x, dc) * cmask_ref[kw]
        for kh in range(k):
            tap = _lshift(col, (kh - pad) * w)
            term = tap * wdw_ref[kh * k + kw]
            acc = term if acc is None else acc + term

    out = jnp.dot(wpw_ref[...], acc.astype(jnp.bfloat16),
                  preferred_element_type=jnp.float32)
    o_ref[0] = out


def kernel(x_nchw, w_dw, w_pw):
    n, cin, h, w = x_nchw.shape
    k = w_dw.shape[2]
    pad = (k - 1) // 2
    cout = w_pw.shape[0]
    hw = h * w

    wdw_v = jnp.transpose(w_dw.reshape(cin, k * k), (1, 0)).reshape(k * k, cin, 1)
    wdw_v = wdw_v.astype(jnp.float32)
    wpw_v = w_pw.reshape(cout, cin).astype(jnp.bfloat16)

    owi = jnp.arange(w)[None, :] + jnp.arange(k)[:, None] - pad
    cmask = ((owi >= 0) & (owi < w)).astype(jnp.float32)
    cmask = jnp.broadcast_to(cmask[:, None, :], (k, h, w)).reshape(k, 1, hw)

    x3 = x_nchw.reshape(n, cin, hw)

    body = functools.partial(_fused_kernel, k=k, pad=pad, w=w)

    out3 = pl.pallas_call(
        body,
        out_shape=jax.ShapeDtypeStruct((n, cout, hw), x_nchw.dtype),
        grid=(n,),
        in_specs=[
            pl.BlockSpec((k * k, cin, 1), lambda b: (0, 0, 0)),
            pl.BlockSpec((cout, cin), lambda b: (0, 0)),
            pl.BlockSpec((k, 1, hw), lambda b: (0, 0, 0)),
            pl.BlockSpec((1, cin, hw), lambda b: (b, 0, 0)),
        ],
        out_specs=pl.BlockSpec((1, cout, hw), lambda b: (b, 0, 0)),
        compiler_params=pltpu.CompilerParams(
            dimension_semantics=("parallel",)),
    )(wdw_v, wpw_v, cmask, x3)

    return out3.reshape(n, cout, h, w)


# speedup vs baseline: 6.5394x; 1.4732x over previous
import functools

import jax
import jax.numpy as jnp
from jax.experimental import pallas as pl
from jax.experimental.pallas import tpu as pltpu


def _lshift(x, d):
    if d == 0:
        return x
    n = x.shape[-1]
    zeros = jnp.zeros(x.shape[:-1] + (abs(d),), x.dtype)
    if d > 0:
        return jnp.concatenate([x[..., d:], zeros], axis=-1)
    return jnp.concatenate([zeros, x[..., :n + d]], axis=-1)


def _fused_kernel(wdw_ref, wpw_ref, cmask_ref, x_ref, o_ref, *, k, pad, w):
    m = x_ref.shape[0]
    x = x_ref[...].astype(jnp.bfloat16)

    cols = []
    for kw in range(k):
        dc = kw - pad
        cols.append(x if dc == 0 else _lshift(x, dc) * cmask_ref[kw])

    acc = None
    for kh in range(k):
        term = None
        for kw in range(k):
            t = cols[kw] * wdw_ref[kh * k + kw]
            term = t if term is None else term + t
        term = _lshift(term, (kh - pad) * w)
        acc = term if acc is None else acc + term

    wpw = wpw_ref[...]
    for i in range(m):
        o_ref[i] = jnp.dot(wpw, acc[i], preferred_element_type=jnp.float32)


def kernel(x_nchw, w_dw, w_pw):
    n, cin, h, w = x_nchw.shape
    k = w_dw.shape[2]
    pad = (k - 1) // 2
    cout = w_pw.shape[0]
    hw = h * w

    imgs = 4
    while n % imgs:
        imgs //= 2

    wdw_v = jnp.transpose(w_dw.reshape(cin, k * k), (1, 0)).reshape(k * k, cin, 1)
    wdw_v = wdw_v.astype(jnp.bfloat16)
    wpw_v = w_pw.reshape(cout, cin).astype(jnp.bfloat16)

    owi = jnp.arange(w)[None, :] + jnp.arange(k)[:, None] - pad
    cmask = ((owi >= 0) & (owi < w)).astype(jnp.bfloat16)
    cmask = jnp.broadcast_to(cmask[:, None, :], (k, h, w)).reshape(k, 1, hw)

    x3 = x_nchw.reshape(n, cin, hw)

    body = functools.partial(_fused_kernel, k=k, pad=pad, w=w)

    out3 = pl.pallas_call(
        body,
        out_shape=jax.ShapeDtypeStruct((n, cout, hw), x_nchw.dtype),
        grid=(n // imgs,),
        in_specs=[
            pl.BlockSpec((k * k, cin, 1), lambda b: (0, 0, 0)),
            pl.BlockSpec((cout, cin), lambda b: (0, 0)),
            pl.BlockSpec((k, 1, hw), lambda b: (0, 0, 0)),
            pl.BlockSpec((imgs, cin, hw), lambda b: (b, 0, 0)),
        ],
        out_specs=pl.BlockSpec((imgs, cout, hw), lambda b: (b, 0, 0)),
        compiler_params=pltpu.CompilerParams(
            dimension_semantics=("parallel",)),
    )(wdw_v, wpw_v, cmask, x3)

    return out3.reshape(n, cout, h, w)


# speedup vs baseline: 29.0198x; 4.4377x over previous
import functools

import jax
import jax.numpy as jnp
from jax.experimental import pallas as pl
from jax.experimental.pallas import tpu as pltpu


def _sep_kernel(wdw_ref, wpw_ref, x_ref, o_ref, *, k, pad, h, w):
    m, cin = x_ref.shape[1], x_ref.shape[2]
    cout = o_ref.shape[2]

    xb = x_ref[...].astype(jnp.bfloat16).reshape(h, w, m, cin)
    zc = jnp.zeros((h, pad, m, cin), jnp.bfloat16)
    zr = jnp.zeros((pad, w + 2 * pad, m, cin), jnp.bfloat16)
    xq = jnp.concatenate([zc, xb, zc], axis=1)
    xq = jnp.concatenate([zr, xq, zr], axis=0)

    acc = None
    for kh in range(k):
        for kw in range(k):
            tap = xq[kh:kh + h, kw:kw + w]
            term = tap * wdw_ref[kh * k + kw]
            acc = term if acc is None else acc + term

    dw2 = acc.reshape(h * w * m, cin)
    out = jnp.dot(dw2, wpw_ref[...], preferred_element_type=jnp.float32)
    o_ref[...] = out.reshape(h * w, m, cout)


def kernel(x_nchw, w_dw, w_pw):
    n, cin, h, w = x_nchw.shape
    k = w_dw.shape[2]
    pad = (k - 1) // 2
    cout = w_pw.shape[0]
    hw = h * w

    m = 16
    while n % m:
        m //= 2

    wdw_v = jnp.transpose(w_dw.reshape(cin, k * k), (1, 0)).reshape(k * k, 1, cin)
    wdw_v = wdw_v.astype(jnp.bfloat16)
    wpw_v = jnp.transpose(w_pw.reshape(cout, cin), (1, 0)).astype(jnp.bfloat16)

    xt = jnp.transpose(x_nchw, (2, 3, 0, 1)).reshape(hw, n, cin)

    body = functools.partial(_sep_kernel, k=k, pad=pad, h=h, w=w)

    out3 = pl.pallas_call(
        body,
        out_shape=jax.ShapeDtypeStruct((hw, n, cout), x_nchw.dtype),
        grid=(n // m,),
        in_specs=[
            pl.BlockSpec((k * k, 1, cin), lambda b: (0, 0, 0)),
            pl.BlockSpec((cin, cout), lambda b: (0, 0)),
            pl.BlockSpec((hw, m, cin), lambda b: (0, b, 0)),
        ],
        out_specs=pl.BlockSpec((hw, m, cout), lambda b: (0, b, 0)),
        compiler_params=pltpu.CompilerParams(
            dimension_semantics=("parallel",),
            vmem_limit_bytes=56 * 2 ** 20),
    )(wdw_v, wpw_v, xt)

    return jnp.transpose(out3.reshape(h, w, n, cout), (2, 3, 0, 1))
